# baseline (speedup 1.0000x reference)
import functools

import jax
import jax.numpy as jnp
from jax import lax
from jax.experimental import pallas as pl
from jax.experimental.pallas import tpu as pltpu
from jax.experimental.pallas import tpu_sc as plsc

N_NODES = 10000
N_EDGES = 160000
IN_CH = 256
OUT_CH = 256
HALF = 128
NC = 2
NS = 16
KB = 125
KC = 125
A_BATCHES = N_EDGES // (NC * NS * KB)
NQ = 4
QB = 20
NPAD = 10240
SEG = NPAD // NS
RPT = 624
TAIL = N_NODES - NS * RPT

_MESH = dict(core_axis_name="c", subcore_axis_name="s")


def _deg_body(dsta, ones_h, zern, out, idx_v, ones_v, cbuf, acc, sem):
    c = lax.axis_index("c")
    s = lax.axis_index("s")
    w = s * NC + c
    pltpu.sync_copy(dsta.at[w], idx_v)
    pltpu.sync_copy(ones_h, ones_v)
    pltpu.sync_copy(zern, cbuf)
    pltpu.sync_copy(cbuf, acc.at[pl.ds(s * SEG, SEG)])
    plsc.subcore_barrier()

    @pl.loop(0, A_BATCHES)
    def _(j):
        pltpu.sync_copy(ones_v, acc.at[idx_v.at[j]], add=True)

    plsc.subcore_barrier()
    pltpu.sync_copy(acc.at[pl.ds(s * SEG, SEG)], cbuf)
    pltpu.sync_copy(cbuf, out.at[c, pl.ds(s * SEG, SEG)])
    del sem


def _deg_call(dsta, ones_h, zern):
    mesh = plsc.VectorSubcoreMesh(**_MESH)
    fn = functools.partial(
        pl.kernel,
        out_type=jax.ShapeDtypeStruct((NC, NPAD), jnp.float32),
        mesh=mesh,
        scratch_types=[
            pltpu.VMEM((A_BATCHES, KB), jnp.int32),
            pltpu.VMEM((KB,), jnp.float32),
            pltpu.VMEM((SEG,), jnp.float32),
            pltpu.VMEM_SHARED((NPAD,), jnp.float32),
            pltpu.SemaphoreType.DMA,
        ],
    )(_deg_body)
    return fn(dsta, ones_h, zern)


def _agg_body(g_flat, srcq, dstq, out,
              is0, id0, is1, id1, rows0, rows1, acc,
              semi0, semi1, semg0, semg1):
    c = lax.axis_index("c")
    s = lax.axis_index("s")
    pltpu.sync_copy(g_flat.at[pl.ds(c * N_NODES + s * RPT, RPT)],
                    acc.at[pl.ds(s * RPT, RPT)])

    @pl.when(s == 0)
    def _():
        pltpu.sync_copy(g_flat.at[pl.ds(c * N_NODES + NS * RPT, TAIL)],
                        acc.at[pl.ds(NS * RPT, TAIL)])

    plsc.subcore_barrier()

    pltpu.sync_copy(srcq.at[c, s, 0], is0)
    pltpu.sync_copy(dstq.at[s, 0], id0)
    pltpu.async_copy(srcq.at[c, s, 1], is1, semi1)
    pltpu.async_copy(dstq.at[s, 1], id1, semi1)

    def chunk(q, is_, id_, semi, first):
        if not first:
            @pl.when(q > 0)
            def _():
                pltpu.make_async_copy(srcq.at[c, s, q], is_, semi).wait()
                pltpu.make_async_copy(dstq.at[s, q], id_, semi).wait()
        else:
            pltpu.make_async_copy(srcq.at[c, s, q], is_, semi).wait()
            pltpu.make_async_copy(dstq.at[s, q], id_, semi).wait()
        pltpu.async_copy(g_flat.at[is_.at[0]], rows0, semg0)
        pltpu.async_copy(g_flat.at[is_.at[1]], rows1, semg1)
        for b in range(QB):
            rb, sg = (rows0, semg0) if b % 2 == 0 else (rows1, semg1)
            pltpu.make_async_copy(g_flat.at[is_.at[b]], rb, sg).wait()
            pltpu.sync_copy(rb, acc.at[id_.at[b]], add=True)
            if b + 2 < QB:
                pltpu.async_copy(g_flat.at[is_.at[b + 2]], rb, sg)
        @pl.when(q + 2 < NQ)
        def _():
            pltpu.async_copy(srcq.at[c, s, q + 2], is_, semi)
            pltpu.async_copy(dstq.at[s, q + 2], id_, semi)

    @pl.loop(0, NQ, step=2)
    def _(q):
        chunk(q, is0, id0, semi0, False)
        chunk(q + 1, is1, id1, semi1, True)

    plsc.subcore_barrier()
    pltpu.sync_copy(acc.at[pl.ds(s * RPT, RPT)], out.at[c, pl.ds(s * RPT, RPT)])

    @pl.when(s == 0)
    def _():
        pltpu.sync_copy(acc.at[pl.ds(NS * RPT, TAIL)],
                        out.at[c, pl.ds(NS * RPT, TAIL)])


def _agg_call(g_flat, srcq, dstq):
    mesh = plsc.VectorSubcoreMesh(**_MESH)
    fn = functools.partial(
        pl.kernel,
        out_type=jax.ShapeDtypeStruct((NC, N_NODES, HALF), jnp.float32),
        mesh=mesh,
        scratch_types=[
            pltpu.VMEM((QB, KC), jnp.int32),
            pltpu.VMEM((QB, KC), jnp.int32),
            pltpu.VMEM((QB, KC), jnp.int32),
            pltpu.VMEM((QB, KC), jnp.int32),
            pltpu.VMEM((KC, HALF), jnp.float32),
            pltpu.VMEM((KC, HALF), jnp.float32),
            pltpu.VMEM_SHARED((N_NODES, HALF), jnp.float32),
            pltpu.SemaphoreType.DMA,
            pltpu.SemaphoreType.DMA,
            pltpu.SemaphoreType.DMA,
            pltpu.SemaphoreType.DMA,
        ],
    )(_agg_body)
    return fn(g_flat, srcq, dstq)


_ROWS_BLK = 10000


def _mm_body(x_ref, w_ref, degt_ref, g_ref):
    d = degt_ref[:, 0:1] + degt_ref[:, 1:2] + 1.0
    dinv = lax.rsqrt(d)
    h = jnp.dot(x_ref[...], w_ref[...], preferred_element_type=jnp.float32)
    g = h * dinv
    g_ref[0] = g[:, :HALF]
    g_ref[1] = g[:, HALF:]


def _mm_call(x, W, degt):
    return pl.pallas_call(
        _mm_body,
        grid=(N_NODES // _ROWS_BLK,),
        in_specs=[
            pl.BlockSpec((_ROWS_BLK, IN_CH), lambda i: (i, 0)),
            pl.BlockSpec((IN_CH, OUT_CH), lambda i: (0, 0)),
            pl.BlockSpec((_ROWS_BLK, NC), lambda i: (i, 0)),
        ],
        out_specs=pl.BlockSpec((NC, _ROWS_BLK, HALF), lambda i: (0, i, 0)),
        out_shape=jax.ShapeDtypeStruct((NC, N_NODES, HALF), jnp.float32),
    )(x, W, degt)


def _fin_body(acc_ref, degt_ref, b_ref, o_ref):
    d = degt_ref[:, 0:1] + degt_ref[:, 1:2] + 1.0
    dinv = lax.rsqrt(d)
    o_ref[:, :HALF] = acc_ref[0] * dinv + b_ref[:, :HALF]
    o_ref[:, HALF:] = acc_ref[1] * dinv + b_ref[:, HALF:]


def _fin_call(acc, degt, b2):
    return pl.pallas_call(
        _fin_body,
        grid=(N_NODES // _ROWS_BLK,),
        in_specs=[
            pl.BlockSpec((NC, _ROWS_BLK, HALF), lambda i: (0, i, 0)),
            pl.BlockSpec((_ROWS_BLK, NC), lambda i: (i, 0)),
            pl.BlockSpec((1, OUT_CH), lambda i: (0, 0)),
        ],
        out_specs=pl.BlockSpec((_ROWS_BLK, OUT_CH), lambda i: (i, 0)),
        out_shape=jax.ShapeDtypeStruct((N_NODES, OUT_CH), jnp.float32),
    )(acc, degt, b2)


def kernel(x, edge_index, W, b):
    ei = edge_index.astype(jnp.int32)
    src, dst = ei[0], ei[1]
    dsta = dst.reshape(NC * NS, A_BATCHES, KB)
    srcc = src.reshape(NS, NQ, QB, KC)
    srcq = jnp.stack([srcc, srcc + N_NODES], axis=0)
    dstq = dst.reshape(NS, NQ, QB, KC)
    zern = jnp.zeros((SEG,), jnp.float32)
    ones_h = jnp.ones((KB,), jnp.float32)

    degp = _deg_call(dsta, ones_h, zern)
    degt = degp[:, :N_NODES].T
    g = _mm_call(x, W, degt)
    acc = _agg_call(g.reshape(NC * N_NODES, HALF), srcq, dstq)
    return _fin_call(acc, degt, b.reshape(1, OUT_CH))


# speedup vs baseline: 1.0221x; 1.0221x over previous
import functools

import jax
import jax.numpy as jnp
from jax import lax
from jax.experimental import pallas as pl
from jax.experimental.pallas import tpu as pltpu
from jax.experimental.pallas import tpu_sc as plsc

N_NODES = 10000
N_EDGES = 160000
IN_CH = 256
OUT_CH = 256
HALF = 128
NC = 2
NS = 16
KB = 125
KC = 125
A_BATCHES = N_EDGES // (NC * NS * KB)
NQ = 4
QB = 20
NPAD = 10240
SEG = NPAD // NS
RPT = 624
TAIL = N_NODES - NS * RPT

_MESH = dict(core_axis_name="c", subcore_axis_name="s")


def _deg_body(dsta, ones_h, zern, out, idx_v, ones_v, cbuf, acc, sem):
    c = lax.axis_index("c")
    s = lax.axis_index("s")
    w = s * NC + c
    pltpu.sync_copy(dsta.at[w], idx_v)
    pltpu.sync_copy(ones_h, ones_v)
    pltpu.sync_copy(zern, cbuf)
    pltpu.sync_copy(cbuf, acc.at[pl.ds(s * SEG, SEG)])
    plsc.subcore_barrier()

    @pl.loop(0, A_BATCHES)
    def _(j):
        pltpu.sync_copy(ones_v, acc.at[idx_v.at[j]], add=True)

    plsc.subcore_barrier()
    pltpu.sync_copy(acc.at[pl.ds(s * SEG, SEG)], cbuf)
    pltpu.sync_copy(cbuf, out.at[c, pl.ds(s * SEG, SEG)])
    del sem


def _deg_call(dsta, ones_h, zern):
    mesh = plsc.VectorSubcoreMesh(**_MESH)
    fn = functools.partial(
        pl.kernel,
        out_type=jax.ShapeDtypeStruct((NC, NPAD), jnp.float32),
        mesh=mesh,
        scratch_types=[
            pltpu.VMEM((A_BATCHES, KB), jnp.int32),
            pltpu.VMEM((KB,), jnp.float32),
            pltpu.VMEM((SEG,), jnp.float32),
            pltpu.VMEM_SHARED((NPAD,), jnp.float32),
            pltpu.SemaphoreType.DMA,
        ],
    )(_deg_body)
    return fn(dsta, ones_h, zern)


def _agg_body(g_flat, srcq, dstq, out,
              is0, id0, is1, id1, rows0, rows1, acc,
              semi0, semi1, semg0, semg1):
    c = lax.axis_index("c")
    s = lax.axis_index("s")
    pltpu.sync_copy(g_flat.at[pl.ds(c * N_NODES + s * RPT, RPT)],
                    acc.at[pl.ds(s * RPT, RPT)])

    @pl.when(s == 0)
    def _():
        pltpu.sync_copy(g_flat.at[pl.ds(c * N_NODES + NS * RPT, TAIL)],
                        acc.at[pl.ds(NS * RPT, TAIL)])

    plsc.subcore_barrier()

    pltpu.sync_copy(srcq.at[c, s, 0], is0)
    pltpu.sync_copy(dstq.at[s, 0], id0)
    pltpu.async_copy(srcq.at[c, s, 1], is1, semi1)
    pltpu.async_copy(dstq.at[s, 1], id1, semi1)

    def chunk(q, is_, id_, semi, first):
        if not first:
            @pl.when(q > 0)
            def _():
                pltpu.make_async_copy(srcq.at[c, s, q], is_, semi).wait()
                pltpu.make_async_copy(dstq.at[s, q], id_, semi).wait()
        else:
            pltpu.make_async_copy(srcq.at[c, s, q], is_, semi).wait()
            pltpu.make_async_copy(dstq.at[s, q], id_, semi).wait()
        pltpu.async_copy(g_flat.at[is_.at[0]], rows0, semg0)
        pltpu.async_copy(g_flat.at[is_.at[1]], rows1, semg1)
        for b in range(QB):
            rb, sg = (rows0, semg0) if b % 2 == 0 else (rows1, semg1)
            pltpu.make_async_copy(g_flat.at[is_.at[b]], rb, sg).wait()
            pltpu.sync_copy(rb, acc.at[id_.at[b]], add=True)
            if b + 2 < QB:
                pltpu.async_copy(g_flat.at[is_.at[b + 2]], rb, sg)
        @pl.when(q + 2 < NQ)
        def _():
            pltpu.async_copy(srcq.at[c, s, q + 2], is_, semi)
            pltpu.async_copy(dstq.at[s, q + 2], id_, semi)

    @pl.loop(0, NQ, step=2)
    def _(q):
        chunk(q, is0, id0, semi0, False)
        chunk(q + 1, is1, id1, semi1, True)

    plsc.subcore_barrier()
    pltpu.sync_copy(acc.at[pl.ds(s * RPT, RPT)], out.at[c, pl.ds(s * RPT, RPT)])

    @pl.when(s == 0)
    def _():
        pltpu.sync_copy(acc.at[pl.ds(NS * RPT, TAIL)],
                        out.at[c, pl.ds(NS * RPT, TAIL)])


def _agg_call(g_flat, srcq, dstq):
    mesh = plsc.VectorSubcoreMesh(**_MESH)
    fn = functools.partial(
        pl.kernel,
        out_type=jax.ShapeDtypeStruct((NC, N_NODES, HALF), jnp.float32),
        mesh=mesh,
        scratch_types=[
            pltpu.VMEM((QB, KC), jnp.int32),
            pltpu.VMEM((QB, KC), jnp.int32),
            pltpu.VMEM((QB, KC), jnp.int32),
            pltpu.VMEM((QB, KC), jnp.int32),
            pltpu.VMEM((KC, HALF), jnp.float32),
            pltpu.VMEM((KC, HALF), jnp.float32),
            pltpu.VMEM_SHARED((N_NODES, HALF), jnp.float32),
            pltpu.SemaphoreType.DMA,
            pltpu.SemaphoreType.DMA,
            pltpu.SemaphoreType.DMA,
            pltpu.SemaphoreType.DMA,
        ],
    )(_agg_body)
    return fn(g_flat, srcq, dstq)


_ROWS_BLK = 5000


def _mm_body(x_ref, w_ref, degt_ref, g_ref):
    d = degt_ref[:, 0:1] + degt_ref[:, 1:2] + 1.0
    dinv = lax.rsqrt(d)
    h = jnp.dot(x_ref[...], w_ref[...], preferred_element_type=jnp.float32)
    g = h * dinv
    g_ref[0] = g[:, :HALF]
    g_ref[1] = g[:, HALF:]


def _mm_call(x, W, degt):
    return pl.pallas_call(
        _mm_body,
        grid=(N_NODES // _ROWS_BLK,),
        in_specs=[
            pl.BlockSpec((_ROWS_BLK, IN_CH), lambda i: (i, 0)),
            pl.BlockSpec((IN_CH, OUT_CH), lambda i: (0, 0)),
            pl.BlockSpec((_ROWS_BLK, NC), lambda i: (i, 0)),
        ],
        out_specs=pl.BlockSpec((NC, _ROWS_BLK, HALF), lambda i: (0, i, 0)),
        out_shape=jax.ShapeDtypeStruct((NC, N_NODES, HALF), jnp.float32),
    )(x, W, degt)


def _fin_body(acc_ref, degt_ref, b_ref, o_ref):
    d = degt_ref[:, 0:1] + degt_ref[:, 1:2] + 1.0
    dinv = lax.rsqrt(d)
    o_ref[:, :HALF] = acc_ref[0] * dinv + b_ref[:, :HALF]
    o_ref[:, HALF:] = acc_ref[1] * dinv + b_ref[:, HALF:]


def _fin_call(acc, degt, b2):
    return pl.pallas_call(
        _fin_body,
        grid=(N_NODES // _ROWS_BLK,),
        in_specs=[
            pl.BlockSpec((NC, _ROWS_BLK, HALF), lambda i: (0, i, 0)),
            pl.BlockSpec((_ROWS_BLK, NC), lambda i: (i, 0)),
            pl.BlockSpec((1, OUT_CH), lambda i: (0, 0)),
        ],
        out_specs=pl.BlockSpec((_ROWS_BLK, OUT_CH), lambda i: (i, 0)),
        out_shape=jax.ShapeDtypeStruct((N_NODES, OUT_CH), jnp.float32),
    )(acc, degt, b2)


def kernel(x, edge_index, W, b):
    ei = edge_index.astype(jnp.int32)
    src, dst = ei[0], ei[1]
    dsta = dst.reshape(NC * NS, A_BATCHES, KB)
    srcc = src.reshape(NS, NQ, QB, KC)
    srcq = jnp.stack([srcc, srcc + N_NODES], axis=0)
    dstq = dst.reshape(NS, NQ, QB, KC)
    zern = jnp.zeros((SEG,), jnp.float32)
    ones_h = jnp.ones((KB,), jnp.float32)

    degp = _deg_call(dsta, ones_h, zern)
    degt = degp[:, :N_NODES].T
    g = _mm_call(x, W, degt)
    acc = _agg_call(g.reshape(NC * N_NODES, HALF), srcq, dstq)
    return _fin_call(acc, degt, b.reshape(1, OUT_CH))
